# Initial kernel scaffold; baseline (speedup 1.0000x reference)
import jax
import jax.numpy as jnp
from jax.experimental import pallas as pl


def kernel(x, edge_index, W1, b1, W2, b2, gamma, beta):
    raise NotImplementedError("write your pallas kernel here")


# speedup vs baseline: 11.7960x; 11.7960x over previous
import functools

import jax
import jax.numpy as jnp
from jax import lax
from jax.experimental import pallas as pl
from jax.experimental.pallas import tpu as pltpu
from jax.experimental.pallas import tpu_sc as plsc

N = 10000
E = 320000
D = 128
DH = D // 2

NC = 2
NS = 16
NW = NC * NS
BATCH = 40
NCH_DEG = E // NW // BATCH
NCH_AGG = E // NS // BATCH
N_PAD = 10240
RPT = N_PAD // NS

_MESH = plsc.VectorSubcoreMesh(
    core_axis_name="c", subcore_axis_name="s", num_cores=NC, num_subcores=NS)


@functools.partial(
    pl.kernel,
    out_type=jax.ShapeDtypeStruct((NC, N_PAD, 16), jnp.float32),
    mesh=_MESH,
    scratch_types=[
        pltpu.VMEM((NCH_DEG, BATCH), jnp.int32),
        pltpu.VMEM((BATCH, 16), jnp.float32),
        pltpu.VMEM_SHARED((N_PAD, 16), jnp.float32),
    ],
    compiler_params=pltpu.CompilerParams(use_tc_tiling_on_sc=False),
)
def _deg_kernel(dst_hbm, out_hbm, idx_v, const_v, accum):
    c = lax.axis_index("c")
    s = lax.axis_index("s")
    wid = c * NS + s

    def _fill(i, val):
        const_v[i, :] = jnp.full((16,), val, jnp.float32)
        return val

    lax.fori_loop(0, BATCH, _fill, 0.0)

    def _zero(j, carry):
        pltpu.sync_copy(const_v, accum.at[pl.ds(s * RPT + j * BATCH, BATCH)])
        return carry

    lax.fori_loop(0, RPT // BATCH, _zero, 0)

    lax.fori_loop(0, BATCH, _fill, 1.0)
    pltpu.sync_copy(dst_hbm.at[wid], idx_v)
    plsc.subcore_barrier()

    def _edge(j, carry):
        pltpu.sync_copy(const_v, accum.at[idx_v.at[j]], add=True)
        return carry

    lax.fori_loop(0, NCH_DEG, _edge, 0)
    plsc.subcore_barrier()

    pltpu.sync_copy(accum.at[pl.ds(s * RPT, RPT)],
                    out_hbm.at[c, pl.ds(s * RPT, RPT)])


@functools.partial(
    pl.kernel,
    out_type=jax.ShapeDtypeStruct((NC, N_PAD, DH), jnp.float32),
    mesh=_MESH,
    scratch_types=[
        pltpu.VMEM((NCH_AGG, BATCH), jnp.int32),
        pltpu.VMEM((NCH_AGG, BATCH), jnp.int32),
        pltpu.VMEM((BATCH, DH), jnp.float32),
        pltpu.VMEM((BATCH, DH), jnp.float32),
        pltpu.VMEM_SHARED((N_PAD, DH), jnp.float32),
        pltpu.SemaphoreType.DMA,
        pltpu.SemaphoreType.DMA,
    ],
    compiler_params=pltpu.CompilerParams(use_tc_tiling_on_sc=False),
)
def _agg_kernel(g_hbm, src_hbm, dst_hbm, out_hbm,
                srcv, dstv, rows0, rows1, accum, sem0, sem1):
    c = lax.axis_index("c")
    s = lax.axis_index("s")
    gc = g_hbm.at[c]

    def _fill(i, carry):
        def _lane(k, inner):
            rows0[i, pl.ds(k * 16, 16)] = jnp.zeros((16,), jnp.float32)
            return inner
        return lax.fori_loop(0, DH // 16, _lane, carry)

    lax.fori_loop(0, BATCH, _fill, 0)

    def _zero(j, carry):
        pltpu.sync_copy(rows0, accum.at[pl.ds(s * RPT + j * BATCH, BATCH)])
        return carry

    lax.fori_loop(0, RPT // BATCH, _zero, 0)

    pltpu.sync_copy(src_hbm.at[s], srcv)
    pltpu.sync_copy(dst_hbm.at[s], dstv)
    plsc.subcore_barrier()

    pltpu.async_copy(gc.at[srcv.at[0]], rows0, sem0)

    def _pair(k, carry):
        j0 = 2 * k
        pltpu.make_async_copy(gc.at[srcv.at[j0]], rows0, sem0).wait()
        pltpu.async_copy(gc.at[srcv.at[j0 + 1]], rows1, sem1)
        pltpu.sync_copy(rows0, accum.at[dstv.at[j0]], add=True)
        pltpu.make_async_copy(gc.at[srcv.at[j0 + 1]], rows1, sem1).wait()

        @pl.when(k < NCH_AGG // 2 - 1)
        def _prefetch():
            pltpu.async_copy(gc.at[srcv.at[j0 + 2]], rows0, sem0)

        pltpu.sync_copy(rows1, accum.at[dstv.at[j0 + 1]], add=True)
        return carry

    lax.fori_loop(0, NCH_AGG // 2, _pair, 0)
    plsc.subcore_barrier()

    pltpu.sync_copy(accum.at[pl.ds(s * RPT, RPT)],
                    out_hbm.at[c, pl.ds(s * RPT, RPT)])


_BR = 2000


def _dis_from_deg(degb):
    deg = degb[0, :, 0:1] + degb[1, :, 0:1] + 1.0
    return lax.rsqrt(deg)


def _halves(h):
    return h[:, :DH], h[:, DH:]


def _mm1_body(x_ref, w_ref, degb_ref, g_ref):
    dis = _dis_from_deg(degb_ref[...])
    g = dis * jnp.dot(x_ref[...], w_ref[...], preferred_element_type=jnp.float32)
    lo, hi = _halves(g)
    g_ref[0] = lo
    g_ref[1] = hi


def _mm2_body(aggp_ref, g_ref, degb_ref, b_ref, w_ref, g2_ref):
    dis = _dis_from_deg(degb_ref[...])
    agg = jnp.concatenate([aggp_ref[0] + g_ref[0], aggp_ref[1] + g_ref[1]],
                          axis=-1)
    z = jax.nn.relu(dis * agg + b_ref[...])
    g2 = dis * jnp.dot(z, w_ref[...], preferred_element_type=jnp.float32)
    lo, hi = _halves(g2)
    g2_ref[0] = lo
    g2_ref[1] = hi


def _ln_body(aggp_ref, g_ref, degb_ref, b_ref, gamma_ref, beta_ref, o_ref):
    dis = _dis_from_deg(degb_ref[...])
    agg = jnp.concatenate([aggp_ref[0] + g_ref[0], aggp_ref[1] + g_ref[1]],
                          axis=-1)
    h = dis * agg + b_ref[...]
    mean = jnp.mean(h, axis=-1, keepdims=True)
    var = jnp.mean((h - mean) ** 2, axis=-1, keepdims=True)
    o_ref[...] = (h - mean) * lax.rsqrt(var + 1e-5) * gamma_ref[...] + beta_ref[...]


_ROWS = pl.BlockSpec((_BR, D), lambda i: (i, 0))
_HALF = pl.BlockSpec((NC, _BR, DH), lambda i: (0, i, 0))
_DEGB = pl.BlockSpec((NC, _BR, 16), lambda i: (0, i, 0))
_VEC = pl.BlockSpec((1, D), lambda i: (0, 0))
_FULL = pl.BlockSpec((D, D), lambda i: (0, 0))
_G_SHAPE = jax.ShapeDtypeStruct((NC, N, DH), jnp.float32)


def _mm1(x, w1, degp):
    return pl.pallas_call(
        _mm1_body,
        grid=(N // _BR,),
        in_specs=[_ROWS, _FULL, _DEGB],
        out_specs=_HALF,
        out_shape=_G_SHAPE,
    )(x, w1, degp)


def _mm2(aggp, g, degp, b1, w2):
    return pl.pallas_call(
        _mm2_body,
        grid=(N // _BR,),
        in_specs=[_HALF, _HALF, _DEGB, _VEC, _FULL],
        out_specs=_HALF,
        out_shape=_G_SHAPE,
    )(aggp, g, degp, b1, w2)


def _ln(aggp, g, degp, b2, gamma, beta):
    return pl.pallas_call(
        _ln_body,
        grid=(N // _BR,),
        in_specs=[_HALF, _HALF, _DEGB, _VEC, _VEC, _VEC],
        out_specs=_ROWS,
        out_shape=jax.ShapeDtypeStruct((N, D), jnp.float32),
    )(aggp, g, degp, b2, gamma, beta)


def kernel(x, edge_index, W1, b1, W2, b2, gamma, beta):
    src_a = edge_index[0].reshape(NS, NCH_AGG, BATCH)
    dst_a = edge_index[1].reshape(NS, NCH_AGG, BATCH)
    dst_d = edge_index[1].reshape(NW, NCH_DEG, BATCH)

    degp = _deg_kernel(dst_d)
    g1 = _mm1(x, W1, degp)
    agg1 = _agg_kernel(g1, src_a, dst_a)
    g2 = _mm2(agg1, g1, degp, b1.reshape(1, D), W2)
    agg2 = _agg_kernel(g2, src_a, dst_a)
    return _ln(agg2, g2, degp, b2.reshape(1, D),
               gamma.reshape(1, D), beta.reshape(1, D))


# speedup vs baseline: 30.7511x; 2.6069x over previous
import functools

import jax
import jax.numpy as jnp
from jax import lax
from jax.experimental import pallas as pl
from jax.experimental.pallas import tpu as pltpu
from jax.experimental.pallas import tpu_sc as plsc

N = 10000
E = 320000
D = 128
DH = D // 2

NC = 2
NS = 16
NW = NC * NS
BATCH = 40
BATCH_A = 80
NBUF = 5
NCH_DEG = E // NW // BATCH
NCH_AGG = E // NS // BATCH_A
N_PAD = 10240
RPT = N_PAD // NS

_MESH = plsc.VectorSubcoreMesh(
    core_axis_name="c", subcore_axis_name="s", num_cores=NC, num_subcores=NS)


@functools.partial(
    pl.kernel,
    out_type=jax.ShapeDtypeStruct((NC, N_PAD, 16), jnp.float32),
    mesh=_MESH,
    scratch_types=[
        pltpu.VMEM((NCH_DEG, BATCH), jnp.int32),
        pltpu.VMEM((BATCH, 16), jnp.float32),
        pltpu.VMEM_SHARED((N_PAD, 16), jnp.float32),
    ],
    compiler_params=pltpu.CompilerParams(use_tc_tiling_on_sc=False),
)
def _deg_kernel(dst_hbm, out_hbm, idx_v, const_v, accum):
    c = lax.axis_index("c")
    s = lax.axis_index("s")
    wid = c * NS + s

    def _fill(i, val):
        const_v[i, :] = jnp.full((16,), val, jnp.float32)
        return val

    lax.fori_loop(0, BATCH, _fill, 0.0)

    def _zero(j, carry):
        pltpu.sync_copy(const_v, accum.at[pl.ds(s * RPT + j * BATCH, BATCH)])
        return carry

    lax.fori_loop(0, RPT // BATCH, _zero, 0)

    lax.fori_loop(0, BATCH, _fill, 1.0)
    pltpu.sync_copy(dst_hbm.at[wid], idx_v)
    plsc.subcore_barrier()

    def _edge(j, carry):
        pltpu.sync_copy(const_v, accum.at[idx_v.at[j]], add=True)
        return carry

    lax.fori_loop(0, NCH_DEG, _edge, 0)
    plsc.subcore_barrier()

    pltpu.sync_copy(accum.at[pl.ds(s * RPT, RPT)],
                    out_hbm.at[c, pl.ds(s * RPT, RPT)])


@functools.partial(
    pl.kernel,
    out_type=jax.ShapeDtypeStruct((NC, N_PAD, DH), jnp.float32),
    mesh=_MESH,
    scratch_types=[
        pltpu.VMEM((NCH_AGG, BATCH_A), jnp.int32),
        pltpu.VMEM((NCH_AGG, BATCH_A), jnp.int32),
        [pltpu.VMEM((BATCH_A, DH), jnp.float32) for _ in range(NBUF)],
        pltpu.VMEM_SHARED((N_PAD, DH), jnp.float32),
        [pltpu.SemaphoreType.DMA for _ in range(NBUF)],
        [pltpu.SemaphoreType.DMA for _ in range(NBUF)],
    ],
    compiler_params=pltpu.CompilerParams(use_tc_tiling_on_sc=False),
)
def _agg_kernel(g_hbm, src_hbm, dst_hbm, out_hbm,
                srcv, dstv, bufs, accum, gsem, ssem):
    c = lax.axis_index("c")
    s = lax.axis_index("s")
    gc = g_hbm.at[c]

    def _fill(i, carry):
        def _lane(k, inner):
            bufs[0][i, pl.ds(k * 16, 16)] = jnp.zeros((16,), jnp.float32)
            return inner
        return lax.fori_loop(0, DH // 16, _lane, carry)

    lax.fori_loop(0, BATCH_A, _fill, 0)

    def _zero(j, carry):
        pltpu.sync_copy(bufs[0], accum.at[pl.ds(s * RPT + j * BATCH_A, BATCH_A)])
        return carry

    lax.fori_loop(0, RPT // BATCH_A, _zero, 0)

    pltpu.sync_copy(src_hbm.at[s], srcv)
    pltpu.sync_copy(dst_hbm.at[s], dstv)
    plsc.subcore_barrier()

    def _gather(j, b):
        pltpu.async_copy(gc.at[srcv.at[j]], bufs[b], gsem[b])

    for b in range(NBUF - 1):
        _gather(b, b)

    def _step(k, carry):
        for b in range(NBUF):
            j = NBUF * k + b
            bn = (b + NBUF - 1) % NBUF
            pltpu.make_async_copy(gc.at[srcv.at[j]], bufs[b], gsem[b]).wait()
            desc = pltpu.make_async_copy(bufs[b], accum.at[dstv.at[j]], ssem[b])
            desc.start(add=True)
            if b == 0:
                @pl.when(k >= 1)
                def _drain():
                    pltpu.make_async_copy(
                        bufs[bn], accum.at[dstv.at[j]], ssem[bn]).wait()
                _gather(j + NBUF - 1, bn)
            else:
                @pl.when(k < NCH_AGG // NBUF - 1)
                def _refill():
                    pltpu.make_async_copy(
                        bufs[bn], accum.at[dstv.at[j]], ssem[bn]).wait()
                    _gather(j + NBUF - 1, bn)
        return carry

    lax.fori_loop(0, NCH_AGG // NBUF, _step, 0)

    for b in range(NBUF):
        pltpu.make_async_copy(bufs[b], accum.at[dstv.at[0]], ssem[b]).wait()
    plsc.subcore_barrier()

    pltpu.sync_copy(accum.at[pl.ds(s * RPT, RPT)],
                    out_hbm.at[c, pl.ds(s * RPT, RPT)])


_BR = 2000


def _dis_from_deg(degb):
    deg = degb[0, :, 0:1] + degb[1, :, 0:1] + 1.0
    return lax.rsqrt(deg)


def _halves(h):
    return h[:, :DH], h[:, DH:]


def _mm1_body(x_ref, w_ref, degb_ref, g_ref):
    dis = _dis_from_deg(degb_ref[...])
    g = dis * jnp.dot(x_ref[...], w_ref[...], preferred_element_type=jnp.float32)
    lo, hi = _halves(g)
    g_ref[0] = lo
    g_ref[1] = hi


def _mm2_body(aggp_ref, g_ref, degb_ref, b_ref, w_ref, g2_ref):
    dis = _dis_from_deg(degb_ref[...])
    agg = jnp.concatenate([aggp_ref[0] + g_ref[0], aggp_ref[1] + g_ref[1]],
                          axis=-1)
    z = jax.nn.relu(dis * agg + b_ref[...])
    g2 = dis * jnp.dot(z, w_ref[...], preferred_element_type=jnp.float32)
    lo, hi = _halves(g2)
    g2_ref[0] = lo
    g2_ref[1] = hi


def _ln_body(aggp_ref, g_ref, degb_ref, b_ref, gamma_ref, beta_ref, o_ref):
    dis = _dis_from_deg(degb_ref[...])
    agg = jnp.concatenate([aggp_ref[0] + g_ref[0], aggp_ref[1] + g_ref[1]],
                          axis=-1)
    h = dis * agg + b_ref[...]
    mean = jnp.mean(h, axis=-1, keepdims=True)
    var = jnp.mean((h - mean) ** 2, axis=-1, keepdims=True)
    o_ref[...] = (h - mean) * lax.rsqrt(var + 1e-5) * gamma_ref[...] + beta_ref[...]


_ROWS = pl.BlockSpec((_BR, D), lambda i: (i, 0))
_HALF = pl.BlockSpec((NC, _BR, DH), lambda i: (0, i, 0))
_DEGB = pl.BlockSpec((NC, _BR, 16), lambda i: (0, i, 0))
_VEC = pl.BlockSpec((1, D), lambda i: (0, 0))
_FULL = pl.BlockSpec((D, D), lambda i: (0, 0))
_G_SHAPE = jax.ShapeDtypeStruct((NC, N, DH), jnp.float32)


def _mm1(x, w1, degp):
    return pl.pallas_call(
        _mm1_body,
        grid=(N // _BR,),
        in_specs=[_ROWS, _FULL, _DEGB],
        out_specs=_HALF,
        out_shape=_G_SHAPE,
    )(x, w1, degp)


def _mm2(aggp, g, degp, b1, w2):
    return pl.pallas_call(
        _mm2_body,
        grid=(N // _BR,),
        in_specs=[_HALF, _HALF, _DEGB, _VEC, _FULL],
        out_specs=_HALF,
        out_shape=_G_SHAPE,
    )(aggp, g, degp, b1, w2)


def _ln(aggp, g, degp, b2, gamma, beta):
    return pl.pallas_call(
        _ln_body,
        grid=(N // _BR,),
        in_specs=[_HALF, _HALF, _DEGB, _VEC, _VEC, _VEC],
        out_specs=_ROWS,
        out_shape=jax.ShapeDtypeStruct((N, D), jnp.float32),
    )(aggp, g, degp, b2, gamma, beta)


def kernel(x, edge_index, W1, b1, W2, b2, gamma, beta):
    src_a = edge_index[0].reshape(NS, NCH_AGG, BATCH_A)
    dst_a = edge_index[1].reshape(NS, NCH_AGG, BATCH_A)
    dst_d = edge_index[1].reshape(NW, NCH_DEG, BATCH)

    degp = _deg_kernel(dst_d)
    g1 = _mm1(x, W1, degp)
    agg1 = _agg_kernel(g1, src_a, dst_a)
    g2 = _mm2(agg1, g1, degp, b1.reshape(1, D), W2)
    agg2 = _agg_kernel(g2, src_a, dst_a)
    return _ln(agg2, g2, degp, b2.reshape(1, D),
               gamma.reshape(1, D), beta.reshape(1, D))


# speedup vs baseline: 37.1670x; 1.2086x over previous
import functools

import jax
import jax.numpy as jnp
from jax import lax
from jax.experimental import pallas as pl
from jax.experimental.pallas import tpu as pltpu
from jax.experimental.pallas import tpu_sc as plsc

N = 10000
E = 320000
D = 128
DH = D // 2

NC = 2
NS = 16
BATCH = 80
NBUF = 5
NCH_AGG = E // NS // BATCH
NCH_DEG = NCH_AGG // NC
N_PAD = 10240
RPT = N_PAD // NS

_MESH = plsc.VectorSubcoreMesh(
    core_axis_name="c", subcore_axis_name="s", num_cores=NC, num_subcores=NS)
_SC_PARAMS = pltpu.CompilerParams(use_tc_tiling_on_sc=False)


@functools.partial(
    pl.kernel,
    out_type=jax.ShapeDtypeStruct((N_PAD, D), jnp.float32),
    mesh=_MESH,
    scratch_types=[
        pltpu.VMEM((NCH_DEG, BATCH), jnp.int32),
        pltpu.VMEM((BATCH, 16), jnp.float32),
        pltpu.VMEM_SHARED((N_PAD, 16), jnp.float32),
    ],
    compiler_params=_SC_PARAMS,
)
def _deg_kernel(dst_hbm, out_hbm, idx_v, const_v, accum):
    c = lax.axis_index("c")
    s = lax.axis_index("s")

    def _fill(i, val):
        const_v[i, :] = jnp.full((16,), val, jnp.float32)
        return val

    lax.fori_loop(0, BATCH, _fill, 0.0)

    def _zero(j, carry):
        pltpu.sync_copy(const_v, accum.at[pl.ds(s * RPT + j * BATCH, BATCH)])
        return carry

    lax.fori_loop(0, RPT // BATCH, _zero, 0)

    lax.fori_loop(0, BATCH, _fill, 1.0)
    pltpu.sync_copy(dst_hbm.at[s, pl.ds(c * NCH_DEG, NCH_DEG)], idx_v)
    plsc.subcore_barrier()

    def _edge(j, carry):
        pltpu.sync_copy(const_v, accum.at[idx_v.at[j]], add=True)
        return carry

    lax.fori_loop(0, NCH_DEG, _edge, 0)
    plsc.subcore_barrier()

    pltpu.sync_copy(accum.at[pl.ds(s * RPT, RPT)],
                    out_hbm.at[pl.ds(s * RPT, RPT), pl.ds(c * DH, 16)])


@functools.partial(
    pl.kernel,
    out_type=jax.ShapeDtypeStruct((N_PAD, D), jnp.float32),
    mesh=_MESH,
    scratch_types=[
        pltpu.VMEM((NCH_AGG, BATCH), jnp.int32),
        pltpu.VMEM((NCH_AGG, BATCH), jnp.int32),
        [pltpu.VMEM((BATCH, DH), jnp.float32) for _ in range(NBUF)],
        pltpu.VMEM_SHARED((N_PAD, DH), jnp.float32),
        [pltpu.SemaphoreType.DMA for _ in range(NBUF)],
        [pltpu.SemaphoreType.DMA for _ in range(NBUF)],
    ],
    compiler_params=_SC_PARAMS,
)
def _agg_kernel(g_hbm, src_hbm, dst_hbm, out_hbm,
                srcv, dstv, bufs, accum, gsem, ssem):
    c = lax.axis_index("c")
    s = lax.axis_index("s")
    gc = g_hbm

    def _fill(i, carry):
        def _lane(k, inner):
            bufs[0][i, pl.ds(k * 16, 16)] = jnp.zeros((16,), jnp.float32)
            return inner
        return lax.fori_loop(0, DH // 16, _lane, carry)

    lax.fori_loop(0, BATCH, _fill, 0)

    def _zero(j, carry):
        pltpu.sync_copy(bufs[0], accum.at[pl.ds(s * RPT + j * BATCH, BATCH)])
        return carry

    lax.fori_loop(0, RPT // BATCH, _zero, 0)

    pltpu.sync_copy(src_hbm.at[s], srcv)
    pltpu.sync_copy(dst_hbm.at[s], dstv)

    def _bias(j, carry):
        def _lane(l, inner):
            sl = pl.ds(l * 16, 16)
            srcv[j, sl] = srcv[j, sl] + c
            return inner
        return lax.fori_loop(0, BATCH // 16, _lane, carry)

    lax.fori_loop(0, NCH_AGG, _bias, 0)
    plsc.subcore_barrier()

    def _gather(j, b):
        pltpu.async_copy(gc.at[srcv.at[j]], bufs[b], gsem[b])

    for b in range(NBUF - 1):
        _gather(b, b)

    def _step(k, carry):
        for b in range(NBUF):
            j = NBUF * k + b
            bn = (b + NBUF - 1) % NBUF
            pltpu.make_async_copy(gc.at[srcv.at[j]], bufs[b], gsem[b]).wait()
            desc = pltpu.make_async_copy(bufs[b], accum.at[dstv.at[j]], ssem[b])
            desc.start(add=True)
            if b == 0:
                @pl.when(k >= 1)
                def _drain():
                    pltpu.make_async_copy(
                        bufs[bn], accum.at[dstv.at[j]], ssem[bn]).wait()
                _gather(j + NBUF - 1, bn)
            else:
                @pl.when(k < NCH_AGG // NBUF - 1)
                def _refill():
                    pltpu.make_async_copy(
                        bufs[bn], accum.at[dstv.at[j]], ssem[bn]).wait()
                    _gather(j + NBUF - 1, bn)
        return carry

    lax.fori_loop(0, NCH_AGG // NBUF, _step, 0)

    for b in range(NBUF):
        pltpu.make_async_copy(bufs[b], accum.at[dstv.at[0]], ssem[b]).wait()
    plsc.subcore_barrier()

    pltpu.sync_copy(accum.at[pl.ds(s * RPT, RPT)],
                    out_hbm.at[pl.ds(s * RPT, RPT), pl.ds(c * DH, DH)])


_BR = 2000


def _dis_from_deg(degb):
    deg = degb[:, 0:1] + degb[:, DH:DH + 1] + 1.0
    return lax.rsqrt(deg)


def _mm1_body(x_ref, w_ref, degb_ref, g_ref):
    dis = _dis_from_deg(degb_ref[...])
    g_ref[...] = dis * jnp.dot(x_ref[...], w_ref[...],
                               preferred_element_type=jnp.float32)


def _mm2_body(agg_ref, g_ref, degb_ref, b_ref, w_ref, g2_ref):
    dis = _dis_from_deg(degb_ref[...])
    z = jax.nn.relu(dis * (agg_ref[...] + g_ref[...]) + b_ref[...])
    g2_ref[...] = dis * jnp.dot(z, w_ref[...],
                                preferred_element_type=jnp.float32)


def _ln_body(agg_ref, g_ref, degb_ref, b_ref, gamma_ref, beta_ref, o_ref):
    dis = _dis_from_deg(degb_ref[...])
    h = dis * (agg_ref[...] + g_ref[...]) + b_ref[...]
    mean = jnp.mean(h, axis=-1, keepdims=True)
    var = jnp.mean((h - mean) ** 2, axis=-1, keepdims=True)
    o_ref[...] = (h - mean) * lax.rsqrt(var + 1e-5) * gamma_ref[...] + beta_ref[...]


_ROWS = pl.BlockSpec((_BR, D), lambda i: (i, 0))
_VEC = pl.BlockSpec((1, D), lambda i: (0, 0))
_FULL = pl.BlockSpec((D, D), lambda i: (0, 0))
_G_SHAPE = jax.ShapeDtypeStruct((N, D), jnp.float32)


def _mm1(x, w1, degp):
    return pl.pallas_call(
        _mm1_body,
        grid=(N // _BR,),
        in_specs=[_ROWS, _FULL, _ROWS],
        out_specs=_ROWS,
        out_shape=_G_SHAPE,
    )(x, w1, degp)


def _mm2(agg, g, degp, b1, w2):
    return pl.pallas_call(
        _mm2_body,
        grid=(N // _BR,),
        in_specs=[_ROWS, _ROWS, _ROWS, _VEC, _FULL],
        out_specs=_ROWS,
        out_shape=_G_SHAPE,
    )(agg, g, degp, b1, w2)


def _ln(agg, g, degp, b2, gamma, beta):
    return pl.pallas_call(
        _ln_body,
        grid=(N // _BR,),
        in_specs=[_ROWS, _ROWS, _ROWS, _VEC, _VEC, _VEC],
        out_specs=_ROWS,
        out_shape=jax.ShapeDtypeStruct((N, D), jnp.float32),
    )(agg, g, degp, b2, gamma, beta)


def kernel(x, edge_index, W1, b1, W2, b2, gamma, beta):
    src2_a = (edge_index[0] * 2).reshape(NS, NCH_AGG, BATCH)
    dst_a = edge_index[1].reshape(NS, NCH_AGG, BATCH)

    degp = _deg_kernel(dst_a)
    g1 = _mm1(x, W1, degp)
    agg1 = _agg_kernel(g1.reshape(2 * N, DH), src2_a, dst_a)
    g2 = _mm2(agg1, g1, degp, b1.reshape(1, D), W2)
    agg2 = _agg_kernel(g2.reshape(2 * N, DH), src2_a, dst_a)
    return _ln(agg2, g2, degp, b2.reshape(1, D),
               gamma.reshape(1, D), beta.reshape(1, D))


# speedup vs baseline: 38.2582x; 1.0294x over previous
import functools

import jax
import jax.numpy as jnp
from jax import lax
from jax.experimental import pallas as pl
from jax.experimental.pallas import tpu as pltpu
from jax.experimental.pallas import tpu_sc as plsc

N = 10000
E = 320000
D = 128
DH = D // 2

NC = 2
NS = 16
BATCH = 80
NBUF = 5
NCH_AGG = E // NS // BATCH
NCH_DEG = NCH_AGG // NC
N_PAD = 10240
RPT = N_PAD // NS

_MESH = plsc.VectorSubcoreMesh(
    core_axis_name="c", subcore_axis_name="s", num_cores=NC, num_subcores=NS)
_SC_PARAMS = pltpu.CompilerParams(use_tc_tiling_on_sc=False)


@functools.partial(
    pl.kernel,
    out_type=jax.ShapeDtypeStruct((N_PAD, D), jnp.float32),
    mesh=_MESH,
    scratch_types=[
        pltpu.VMEM((NCH_DEG, BATCH), jnp.int32),
        pltpu.VMEM((BATCH, 16), jnp.float32),
        pltpu.VMEM_SHARED((N_PAD, 16), jnp.float32),
        pltpu.SemaphoreType.DMA,
    ],
    compiler_params=_SC_PARAMS,
)
def _deg_kernel(dst_hbm, out_hbm, idx_v, const_v, accum, sem):
    c = lax.axis_index("c")
    s = lax.axis_index("s")

    def _fill(i, val):
        const_v[i, :] = jnp.full((16,), val, jnp.float32)
        return val

    lax.fori_loop(0, BATCH, _fill, 0.0)

    def _zero(j, carry):
        pltpu.sync_copy(const_v, accum.at[pl.ds(s * RPT + j * BATCH, BATCH)])
        return carry

    lax.fori_loop(0, RPT // BATCH, _zero, 0)

    lax.fori_loop(0, BATCH, _fill, 1.0)
    pltpu.sync_copy(dst_hbm.at[s, pl.ds(c * NCH_DEG, NCH_DEG)], idx_v)
    plsc.subcore_barrier()

    def _edge(j, carry):
        pltpu.make_async_copy(const_v, accum.at[idx_v.at[j]], sem).start(add=True)
        return carry

    lax.fori_loop(0, NCH_DEG, _edge, 0)

    def _drain(j, carry):
        pltpu.make_async_copy(const_v, accum.at[idx_v.at[0]], sem).wait()
        return carry

    lax.fori_loop(0, NCH_DEG, _drain, 0)
    plsc.subcore_barrier()

    pltpu.sync_copy(accum.at[pl.ds(s * RPT, RPT)],
                    out_hbm.at[pl.ds(s * RPT, RPT), pl.ds(c * DH, 16)])


@functools.partial(
    pl.kernel,
    out_type=jax.ShapeDtypeStruct((N_PAD, D), jnp.float32),
    mesh=_MESH,
    scratch_types=[
        pltpu.VMEM((NCH_AGG, BATCH), jnp.int32),
        pltpu.VMEM((NCH_AGG, BATCH), jnp.int32),
        [pltpu.VMEM((BATCH, DH), jnp.float32) for _ in range(NBUF)],
        pltpu.VMEM_SHARED((N_PAD, DH), jnp.float32),
        [pltpu.SemaphoreType.DMA for _ in range(NBUF)],
        [pltpu.SemaphoreType.DMA for _ in range(NBUF)],
    ],
    compiler_params=_SC_PARAMS,
)
def _agg_kernel(g_hbm, src_hbm, dst_hbm, out_hbm,
                srcv, dstv, bufs, accum, gsem, ssem):
    c = lax.axis_index("c")
    s = lax.axis_index("s")
    gc = g_hbm

    def _fill(i, carry):
        def _lane(k, inner):
            bufs[0][i, pl.ds(k * 16, 16)] = jnp.zeros((16,), jnp.float32)
            return inner
        return lax.fori_loop(0, DH // 16, _lane, carry)

    lax.fori_loop(0, BATCH, _fill, 0)

    def _zero(j, carry):
        pltpu.sync_copy(bufs[0], accum.at[pl.ds(s * RPT + j * BATCH, BATCH)])
        return carry

    lax.fori_loop(0, RPT // BATCH, _zero, 0)

    pltpu.sync_copy(src_hbm.at[s], srcv)
    pltpu.sync_copy(dst_hbm.at[s], dstv)

    def _bias(j, carry):
        def _lane(l, inner):
            sl = pl.ds(l * 16, 16)
            srcv[j, sl] = srcv[j, sl] + c
            return inner
        return lax.fori_loop(0, BATCH // 16, _lane, carry)

    lax.fori_loop(0, NCH_AGG, _bias, 0)
    plsc.subcore_barrier()

    def _gather(j, b):
        pltpu.async_copy(gc.at[srcv.at[j]], bufs[b], gsem[b])

    for b in range(NBUF - 1):
        _gather(b, b)

    def _step(k, carry):
        for b in range(NBUF):
            j = NBUF * k + b
            bn = (b + NBUF - 1) % NBUF
            pltpu.make_async_copy(gc.at[srcv.at[j]], bufs[b], gsem[b]).wait()
            desc = pltpu.make_async_copy(bufs[b], accum.at[dstv.at[j]], ssem[b])
            desc.start(add=True)
            if b == 0:
                @pl.when(k >= 1)
                def _drain():
                    pltpu.make_async_copy(
                        bufs[bn], accum.at[dstv.at[j]], ssem[bn]).wait()
                _gather(j + NBUF - 1, bn)
            else:
                @pl.when(k < NCH_AGG // NBUF - 1)
                def _refill():
                    pltpu.make_async_copy(
                        bufs[bn], accum.at[dstv.at[j]], ssem[bn]).wait()
                    _gather(j + NBUF - 1, bn)
        return carry

    lax.fori_loop(0, NCH_AGG // NBUF, _step, 0)

    for b in range(NBUF):
        pltpu.make_async_copy(bufs[b], accum.at[dstv.at[0]], ssem[b]).wait()
    plsc.subcore_barrier()

    pltpu.sync_copy(accum.at[pl.ds(s * RPT, RPT)],
                    out_hbm.at[pl.ds(s * RPT, RPT), pl.ds(c * DH, DH)])


_BR = 2000


def _dis_from_deg(degb):
    deg = degb[:, 0:1] + degb[:, DH:DH + 1] + 1.0
    return lax.rsqrt(deg)


def _mm1_body(x_ref, w_ref, degb_ref, g_ref):
    dis = _dis_from_deg(degb_ref[...])
    g_ref[...] = dis * jnp.dot(x_ref[...], w_ref[...],
                               preferred_element_type=jnp.float32)


def _mm2_body(agg_ref, g_ref, degb_ref, b_ref, w_ref, g2_ref):
    dis = _dis_from_deg(degb_ref[...])
    z = jax.nn.relu(dis * (agg_ref[...] + g_ref[...]) + b_ref[...])
    g2_ref[...] = dis * jnp.dot(z, w_ref[...],
                                preferred_element_type=jnp.float32)


def _ln_body(agg_ref, g_ref, degb_ref, b_ref, gamma_ref, beta_ref, o_ref):
    dis = _dis_from_deg(degb_ref[...])
    h = dis * (agg_ref[...] + g_ref[...]) + b_ref[...]
    mean = jnp.mean(h, axis=-1, keepdims=True)
    var = jnp.mean((h - mean) ** 2, axis=-1, keepdims=True)
    o_ref[...] = (h - mean) * lax.rsqrt(var + 1e-5) * gamma_ref[...] + beta_ref[...]


_ROWS = pl.BlockSpec((_BR, D), lambda i: (i, 0))
_VEC = pl.BlockSpec((1, D), lambda i: (0, 0))
_FULL = pl.BlockSpec((D, D), lambda i: (0, 0))
_G_SHAPE = jax.ShapeDtypeStruct((N, D), jnp.float32)


def _mm1(x, w1, degp):
    return pl.pallas_call(
        _mm1_body,
        grid=(N // _BR,),
        in_specs=[_ROWS, _FULL, _ROWS],
        out_specs=_ROWS,
        out_shape=_G_SHAPE,
    )(x, w1, degp)


def _mm2(agg, g, degp, b1, w2):
    return pl.pallas_call(
        _mm2_body,
        grid=(N // _BR,),
        in_specs=[_ROWS, _ROWS, _ROWS, _VEC, _FULL],
        out_specs=_ROWS,
        out_shape=_G_SHAPE,
    )(agg, g, degp, b1, w2)


def _ln(agg, g, degp, b2, gamma, beta):
    return pl.pallas_call(
        _ln_body,
        grid=(N // _BR,),
        in_specs=[_ROWS, _ROWS, _ROWS, _VEC, _VEC, _VEC],
        out_specs=_ROWS,
        out_shape=jax.ShapeDtypeStruct((N, D), jnp.float32),
    )(agg, g, degp, b2, gamma, beta)


def kernel(x, edge_index, W1, b1, W2, b2, gamma, beta):
    dst_a = edge_index[1].reshape(NS, NCH_AGG, BATCH)
    src2_a = (lax.optimization_barrier(edge_index)[0] * 2).reshape(
        NS, NCH_AGG, BATCH)

    degp = _deg_kernel(dst_a)
    g1 = _mm1(x, W1, degp)
    agg1 = _agg_kernel(g1.reshape(2 * N, DH), src2_a, dst_a)
    g2 = _mm2(agg1, g1, degp, b1.reshape(1, D), W2)
    agg2 = _agg_kernel(g2.reshape(2 * N, DH), src2_a, dst_a)
    return _ln(agg2, g2, degp, b2.reshape(1, D),
               gamma.reshape(1, D), beta.reshape(1, D))


# speedup vs baseline: 38.8706x; 1.0160x over previous
import functools

import jax
import jax.numpy as jnp
from jax import lax
from jax.experimental import pallas as pl
from jax.experimental.pallas import tpu as pltpu
from jax.experimental.pallas import tpu_sc as plsc

N = 10000
E = 320000
D = 128
DH = D // 2

NC = 2
NS = 16
BATCH = 80
NBUF = 5
NCH_AGG = E // NS // BATCH
NCH_DEG = NCH_AGG // NC
N_PAD = 10240
RPT = N_PAD // NS

_MESH = plsc.VectorSubcoreMesh(
    core_axis_name="c", subcore_axis_name="s", num_cores=NC, num_subcores=NS)
_SC_PARAMS = pltpu.CompilerParams(use_tc_tiling_on_sc=False)


@functools.partial(
    pl.kernel,
    out_type=jax.ShapeDtypeStruct((N_PAD, D), jnp.float32),
    mesh=_MESH,
    scratch_types=[
        pltpu.VMEM((NCH_DEG, BATCH), jnp.int32),
        pltpu.VMEM((BATCH, 16), jnp.float32),
        pltpu.VMEM_SHARED((N_PAD, 16), jnp.float32),
        pltpu.SemaphoreType.DMA,
    ],
    compiler_params=_SC_PARAMS,
)
def _deg_kernel(dst_hbm, out_hbm, idx_v, const_v, accum, sem):
    c = lax.axis_index("c")
    s = lax.axis_index("s")

    def _fill(i, val):
        const_v[i, :] = jnp.full((16,), val, jnp.float32)
        return val

    lax.fori_loop(0, BATCH, _fill, 0.0)

    def _zero(j, carry):
        pltpu.sync_copy(const_v, accum.at[pl.ds(s * RPT + j * BATCH, BATCH)])
        return carry

    lax.fori_loop(0, RPT // BATCH, _zero, 0)

    lax.fori_loop(0, BATCH, _fill, 1.0)
    pltpu.sync_copy(dst_hbm.at[s, pl.ds(c * NCH_DEG, NCH_DEG)], idx_v)
    plsc.subcore_barrier()

    def _edge(j, carry):
        pltpu.make_async_copy(const_v, accum.at[idx_v.at[j]], sem).start(add=True)
        return carry

    lax.fori_loop(0, NCH_DEG, _edge, 0)

    def _drain(j, carry):
        pltpu.make_async_copy(const_v, accum.at[idx_v.at[0]], sem).wait()
        return carry

    lax.fori_loop(0, NCH_DEG, _drain, 0)
    plsc.subcore_barrier()

    pltpu.sync_copy(accum.at[pl.ds(s * RPT, RPT)],
                    out_hbm.at[pl.ds(s * RPT, RPT), pl.ds(c * DH, 16)])


@functools.partial(
    pl.kernel,
    out_type=jax.ShapeDtypeStruct((N_PAD, D), jnp.float32),
    mesh=_MESH,
    scratch_types=[
        pltpu.VMEM((NCH_AGG, BATCH), jnp.int32),
        pltpu.VMEM((NCH_AGG, BATCH), jnp.int32),
        [pltpu.VMEM((BATCH, DH), jnp.float32) for _ in range(NBUF)],
        pltpu.VMEM((BATCH, DH), jnp.float32),
        pltpu.VMEM_SHARED((N_PAD, DH), jnp.float32),
        [pltpu.SemaphoreType.DMA for _ in range(NBUF)],
        [pltpu.SemaphoreType.DMA for _ in range(NBUF)],
        pltpu.SemaphoreType.DMA,
    ],
    compiler_params=_SC_PARAMS,
)
def _agg_kernel(g_hbm, src0_hbm, src1_hbm, dst_hbm, out_hbm,
                srcv, dstv, bufs, zbuf, accum, gsem, ssem, zsem):
    c = lax.axis_index("c")
    s = lax.axis_index("s")
    gc = g_hbm

    @pl.when(c == 0)
    def _stage0():
        pltpu.sync_copy(src0_hbm.at[s], srcv)

    @pl.when(c == 1)
    def _stage1():
        pltpu.sync_copy(src1_hbm.at[s], srcv)

    pltpu.sync_copy(dst_hbm.at[s], dstv)

    def _gather(j, b):
        pltpu.async_copy(gc.at[srcv.at[j]], bufs[b], gsem[b])

    for b in range(NBUF - 1):
        _gather(b, b)

    def _fill(i, carry):
        def _lane(k, inner):
            zbuf[i, pl.ds(k * 16, 16)] = jnp.zeros((16,), jnp.float32)
            return inner
        return lax.fori_loop(0, DH // 16, _lane, carry)

    lax.fori_loop(0, BATCH, _fill, 0)

    def _zero(j, carry):
        pltpu.make_async_copy(
            zbuf, accum.at[pl.ds(s * RPT + j * BATCH, BATCH)], zsem).start()
        return carry

    lax.fori_loop(0, RPT // BATCH, _zero, 0)

    def _zdrain(j, carry):
        pltpu.make_async_copy(
            zbuf, accum.at[pl.ds(s * RPT, BATCH)], zsem).wait()
        return carry

    lax.fori_loop(0, RPT // BATCH, _zdrain, 0)
    plsc.subcore_barrier()

    def _step(k, carry):
        for b in range(NBUF):
            j = NBUF * k + b
            bn = (b + NBUF - 1) % NBUF
            pltpu.make_async_copy(gc.at[srcv.at[j]], bufs[b], gsem[b]).wait()
            desc = pltpu.make_async_copy(bufs[b], accum.at[dstv.at[j]], ssem[b])
            desc.start(add=True)
            if b == 0:
                @pl.when(k >= 1)
                def _drain():
                    pltpu.make_async_copy(
                        bufs[bn], accum.at[dstv.at[j]], ssem[bn]).wait()
                _gather(j + NBUF - 1, bn)
            else:
                @pl.when(k < NCH_AGG // NBUF - 1)
                def _refill():
                    pltpu.make_async_copy(
                        bufs[bn], accum.at[dstv.at[j]], ssem[bn]).wait()
                    _gather(j + NBUF - 1, bn)
        return carry

    lax.fori_loop(0, NCH_AGG // NBUF, _step, 0)

    for b in range(NBUF):
        pltpu.make_async_copy(bufs[b], accum.at[dstv.at[0]], ssem[b]).wait()
    plsc.subcore_barrier()

    pltpu.sync_copy(accum.at[pl.ds(s * RPT, RPT)],
                    out_hbm.at[pl.ds(s * RPT, RPT), pl.ds(c * DH, DH)])


_BR = 2000


def _dis_from_deg(degb):
    deg = degb[:, 0:1] + degb[:, DH:DH + 1] + 1.0
    return lax.rsqrt(deg)


def _mm1_body(x_ref, w_ref, degb_ref, g_ref):
    dis = _dis_from_deg(degb_ref[...])
    g_ref[...] = dis * jnp.dot(x_ref[...], w_ref[...],
                               preferred_element_type=jnp.float32)


def _mm2_body(agg_ref, g_ref, degb_ref, b_ref, w_ref, g2_ref):
    dis = _dis_from_deg(degb_ref[...])
    z = jax.nn.relu(dis * (agg_ref[...] + g_ref[...]) + b_ref[...])
    g2_ref[...] = dis * jnp.dot(z, w_ref[...],
                                preferred_element_type=jnp.float32)


def _ln_body(agg_ref, g_ref, degb_ref, b_ref, gamma_ref, beta_ref, o_ref):
    dis = _dis_from_deg(degb_ref[...])
    h = dis * (agg_ref[...] + g_ref[...]) + b_ref[...]
    mean = jnp.mean(h, axis=-1, keepdims=True)
    var = jnp.mean((h - mean) ** 2, axis=-1, keepdims=True)
    o_ref[...] = (h - mean) * lax.rsqrt(var + 1e-5) * gamma_ref[...] + beta_ref[...]


_ROWS = pl.BlockSpec((_BR, D), lambda i: (i, 0))
_VEC = pl.BlockSpec((1, D), lambda i: (0, 0))
_FULL = pl.BlockSpec((D, D), lambda i: (0, 0))
_G_SHAPE = jax.ShapeDtypeStruct((N, D), jnp.float32)


def _mm1(x, w1, degp):
    return pl.pallas_call(
        _mm1_body,
        grid=(N // _BR,),
        in_specs=[_ROWS, _FULL, _ROWS],
        out_specs=_ROWS,
        out_shape=_G_SHAPE,
    )(x, w1, degp)


def _mm2(agg, g, degp, b1, w2):
    return pl.pallas_call(
        _mm2_body,
        grid=(N // _BR,),
        in_specs=[_ROWS, _ROWS, _ROWS, _VEC, _FULL],
        out_specs=_ROWS,
        out_shape=_G_SHAPE,
    )(agg, g, degp, b1, w2)


def _ln(agg, g, degp, b2, gamma, beta):
    return pl.pallas_call(
        _ln_body,
        grid=(N // _BR,),
        in_specs=[_ROWS, _ROWS, _ROWS, _VEC, _VEC, _VEC],
        out_specs=_ROWS,
        out_shape=jax.ShapeDtypeStruct((N, D), jnp.float32),
    )(agg, g, degp, b2, gamma, beta)


def kernel(x, edge_index, W1, b1, W2, b2, gamma, beta):
    dst_a = edge_index[1].reshape(NS, NCH_AGG, BATCH)
    src0_a = (lax.optimization_barrier(edge_index)[0] * 2).reshape(
        NS, NCH_AGG, BATCH)
    src1_a = src0_a + 1

    degp = _deg_kernel(dst_a)
    g1 = _mm1(x, W1, degp)
    agg1 = _agg_kernel(g1.reshape(2 * N, DH), src0_a, src1_a, dst_a)
    g2 = _mm2(agg1, g1, degp, b1.reshape(1, D), W2)
    agg2 = _agg_kernel(g2.reshape(2 * N, DH), src0_a, src1_a, dst_a)
    return _ln(agg2, g2, degp, b2.reshape(1, D),
               gamma.reshape(1, D), beta.reshape(1, D))
